# baseline (device time: 187915 ns/iter reference)
import jax
import jax.numpy as jnp
from jax import lax
from jax.experimental import pallas as pl
from jax.experimental.pallas import tpu as pltpu

N_DEV = 4
F32 = jnp.float32
BF16 = jnp.bfloat16


def _ring_fused_allreduce(produce, epilogue, bufs):
    (send_a, recv_a, send_b, recv_b, ss_a, rs_a, ss_b, rs_b) = bufs
    my = lax.axis_index("i")
    left = lax.rem(my + N_DEV - 1, N_DEV)
    right = lax.rem(my + 1, N_DEV)
    n_hops = 2 * (N_DEV - 1)

    barrier_sem = pltpu.get_barrier_semaphore()
    for nbr in (left, right):
        pl.semaphore_signal(
            barrier_sem, inc=1,
            device_id=(nbr,), device_id_type=pl.DeviceIdType.MESH,
        )
    send_a[0, :, :] = produce(my, 0)
    send_b[0, :, :] = produce(my, 1)
    pl.semaphore_wait(barrier_sem, 2)

    def mk(h, sbuf, rbuf, ssem, rsem, dst):
        return pltpu.make_async_remote_copy(
            src_ref=sbuf.at[h], dst_ref=rbuf.at[h],
            send_sem=ssem.at[h], recv_sem=rsem.at[h],
            device_id=(dst,), device_id_type=pl.DeviceIdType.MESH,
        )

    for h in range(n_hops):
        rd_a = mk(h, send_a, recv_a, ss_a, rs_a, right)
        rd_b = mk(h, send_b, recv_b, ss_b, rs_b, left)
        rd_a.start()
        rd_b.start()
        if h < N_DEV - 1:
            ca = lax.rem(my + N_DEV - h - 1, N_DEV)
            cb = lax.rem(my + h + 1, N_DEV)
            pa = produce(ca, 0)
            pb = produce(cb, 1)
            rd_a.wait()
            rd_b.wait()
            acc_a = recv_a[h] + pa
            acc_b = recv_b[h] + pb
            send_a[h + 1, :, :] = acc_a
            send_b[h + 1, :, :] = acc_b
            if h == N_DEV - 2:
                epilogue(ca, 0, acc_a)
                epilogue(cb, 1, acc_b)
        else:
            g = h - (N_DEV - 1)
            ca = lax.rem(my + N_DEV - g, N_DEV)
            cb = lax.rem(my + g, N_DEV)
            rd_a.wait()
            rd_b.wait()
            epilogue(ca, 0, recv_a[h])
            epilogue(cb, 1, recv_b[h])
            if g < N_DEV - 2:
                send_a[h + 1, :, :] = recv_a[h]
                send_b[h + 1, :, :] = recv_b[h]


def _scratch(rows, C):
    buf = lambda: pltpu.VMEM((2 * (N_DEV - 1), rows, C), BF16)
    sem = lambda: pltpu.SemaphoreType.DMA((2 * (N_DEV - 1),))
    return [buf(), buf(), buf(), buf(), sem(), sem(), sem(), sem()]


def _attn_out_block(Q, K, V, Wo, x, ga, n_heads, collective_id):
    R, C = x.shape
    rows = R // (2 * N_DEV)
    half = R // 2
    Dh = 128
    scale = 0.08838834764831843

    def body(q_ref, k_ref, v_ref, wo_ref, x_ref, ga_ref, x1_ref, att_buf,
             *bufs):
        def rs(c, d):
            return pl.ds(d * half + c * rows, rows)

        def produce(c, d):
            r = rs(c, d)
            kv = pl.ds(d * half, half)
            for hh in range(n_heads):
                hs = slice(hh * Dh, (hh + 1) * Dh)
                q = q_ref[r, hs]
                k = k_ref[kv, hs]
                s = lax.dot_general(
                    q, k, (((1,), (1,)), ((), ())), preferred_element_type=F32
                ) * scale
                mx = jnp.max(s, axis=-1, keepdims=True)
                e = jnp.exp(s - mx)
                p = e / jnp.sum(e, axis=-1, keepdims=True)
                o = jnp.dot(p.astype(BF16), v_ref[kv, hs],
                            preferred_element_type=F32)
                att_buf[:, hs] = o.astype(BF16)
            return jnp.dot(att_buf[:, :], wo_ref[:, :],
                           preferred_element_type=F32).astype(BF16)

        def epilogue(c, d, chunk):
            r = rs(c, d)
            x1_ref[r, :] = x_ref[r, :] + ga_ref[d] * chunk.astype(F32)

        _ring_fused_allreduce(produce, epilogue, bufs)

    return pl.pallas_call(
        body,
        out_shape=jax.ShapeDtypeStruct((R, C), F32),
        in_specs=[pl.BlockSpec(memory_space=pltpu.VMEM)] * 6,
        out_specs=pl.BlockSpec(memory_space=pltpu.VMEM),
        scratch_shapes=[pltpu.VMEM((rows, n_heads * Dh), BF16)]
        + _scratch(rows, C),
        compiler_params=pltpu.CompilerParams(
            collective_id=collective_id, vmem_limit_bytes=100 * 1024 * 1024
        ),
    )(Q, K, V, Wo, x, ga)


def _ffn_block(x1, W1, W2, sm, shm, gm, collective_id):
    R, C = x1.shape
    rows = R // (2 * N_DEV)
    half = R // 2
    eps = 1e-5

    def body(x1_ref, w1_ref, w2_ref, sm_ref, shm_ref, gm_ref, out_ref, *bufs):
        def rs(c, d):
            return pl.ds(d * half + c * rows, rows)

        def produce(c, d):
            xc = x1_ref[rs(c, d), :]
            m = jnp.mean(xc, axis=-1, keepdims=True)
            cen = xc - m
            v = jnp.mean(cen * cen, axis=-1, keepdims=True)
            xm = cen * lax.rsqrt(v + eps) * (1.0 + sm_ref[d]) + shm_ref[d]
            h = jnp.dot(xm.astype(BF16), w1_ref[:, :], preferred_element_type=F32)
            h = h * jax.nn.sigmoid(h)
            return jnp.dot(
                h.astype(BF16), w2_ref[:, :], preferred_element_type=F32
            ).astype(BF16)

        def epilogue(c, d, chunk):
            r = rs(c, d)
            out_ref[r, :] = x1_ref[r, :] + gm_ref[d] * chunk.astype(F32)

        _ring_fused_allreduce(produce, epilogue, bufs)

    return pl.pallas_call(
        body,
        out_shape=jax.ShapeDtypeStruct((R, C), F32),
        in_specs=[pl.BlockSpec(memory_space=pltpu.VMEM)] * 6,
        out_specs=pl.BlockSpec(memory_space=pltpu.VMEM),
        scratch_shapes=_scratch(rows, C),
        compiler_params=pltpu.CompilerParams(
            collective_id=collective_id, vmem_limit_bytes=100 * 1024 * 1024
        ),
    )(x1, W1, W2, sm, shm, gm)


def kernel(x, Wq, Wk, Wv, Wo, t_emb, W_mod, W_ff1, W_ff2):
    B, S, D = x.shape
    Dh = 128
    Hl = Wq.shape[1] // Dh
    eps = 1e-5

    x = x.astype(F32)
    mod = t_emb.astype(F32) @ W_mod.astype(F32)
    sa, sha, ga, sm, shm, gm = jnp.split(mod, 6, axis=-1)

    def ln(h):
        m = jnp.mean(h, axis=-1, keepdims=True)
        v = jnp.var(h, axis=-1, keepdims=True)
        return (h - m) * lax.rsqrt(v + eps)

    xa = (ln(x) * (1.0 + sa[:, None, :]) + sha[:, None, :]).astype(BF16)
    xa = xa.reshape(B * S, D)
    Q = (xa @ Wq.astype(BF16)).astype(BF16)
    K = (xa @ Wk.astype(BF16)).astype(BF16)
    V = (xa @ Wv.astype(BF16)).astype(BF16)

    x1 = _attn_out_block(Q, K, V, Wo.astype(BF16), x.reshape(B * S, D),
                         ga, Hl, collective_id=0)

    out = _ffn_block(x1, W_ff1.astype(BF16), W_ff2.astype(BF16),
                     sm, shm, gm, collective_id=1)
    return out.reshape(B, S, D)


# device time: 174200 ns/iter; 1.0787x vs baseline; 1.0787x over previous
import jax
import jax.numpy as jnp
from jax import lax
from jax.experimental import pallas as pl
from jax.experimental.pallas import tpu as pltpu

N_DEV = 4
N_HOPS = 2 * (N_DEV - 1)
F32 = jnp.float32
BF16 = jnp.bfloat16
EPS = 1e-5


def _dit_block(Q, K, V, Wo, W1, W2, x, ga, sm, shm, gm, n_heads):
    R, C = x.shape
    rows = R // (2 * N_DEV)
    half = R // 2
    Dh = 128
    scale = 0.08838834764831843

    def body(q_ref, k_ref, v_ref, wo_ref, w1_ref, w2_ref, x_ref,
             ga_ref, sm_ref, shm_ref, gm_ref, out_ref,
             att_buf,
             s1a, r1a, s1b, r1b, s2a, r2a, s2b, r2b,
             ss1a, rs1a, ss1b, rs1b, ss2a, rs2a, ss2b, rs2b):
        my = lax.axis_index("i")
        left = lax.rem(my + N_DEV - 1, N_DEV)
        right = lax.rem(my + 1, N_DEV)

        def rs(c, d):
            return pl.ds(d * half + c * rows, rows)

        def produce_attn(c, d):
            r = rs(c, d)
            kv = pl.ds(d * half, half)
            for hh in range(n_heads):
                hs = slice(hh * Dh, (hh + 1) * Dh)
                q = q_ref[r, hs]
                k = k_ref[kv, hs]
                s = lax.dot_general(
                    q, k, (((1,), (1,)), ((), ())), preferred_element_type=F32
                ) * scale
                mx = jnp.max(s, axis=-1, keepdims=True)
                e = jnp.exp(s - mx)
                p = e / jnp.sum(e, axis=-1, keepdims=True)
                o = jnp.dot(p.astype(BF16), v_ref[kv, hs],
                            preferred_element_type=F32)
                att_buf[:, hs] = o.astype(BF16)
            return jnp.dot(att_buf[:, :], wo_ref[:, :],
                           preferred_element_type=F32).astype(BF16)

        def epilogue1(c, d, chunk):
            r = rs(c, d)
            out_ref[r, :] = x_ref[r, :] + ga_ref[d] * chunk.astype(F32)

        def produce_ffn(c, d):
            xc = out_ref[rs(c, d), :]
            m = jnp.mean(xc, axis=-1, keepdims=True)
            cen = xc - m
            v = jnp.mean(cen * cen, axis=-1, keepdims=True)
            xm = cen * lax.rsqrt(v + EPS) * (1.0 + sm_ref[d]) + shm_ref[d]
            h = jnp.dot(xm.astype(BF16), w1_ref[:, :],
                        preferred_element_type=F32)
            h = h * jax.nn.sigmoid(h)
            return jnp.dot(
                h.astype(BF16), w2_ref[:, :], preferred_element_type=F32
            ).astype(BF16)

        def epilogue2(c, d, chunk):
            r = rs(c, d)
            out_ref[r, :] = out_ref[r, :] + gm_ref[d] * chunk.astype(F32)

        def mk(h, src_slot, sbuf, rbuf, ssem, rsem, dst):
            return pltpu.make_async_remote_copy(
                src_ref=sbuf.at[src_slot] if src_slot < N_DEV else rbuf.at[src_slot - 1],
                dst_ref=rbuf.at[h],
                send_sem=ssem.at[h], recv_sem=rsem.at[h],
                device_id=(dst,), device_id_type=pl.DeviceIdType.MESH,
            )

        barrier_sem = pltpu.get_barrier_semaphore()
        for nbr in (left, right):
            pl.semaphore_signal(
                barrier_sem, inc=1,
                device_id=(nbr,), device_id_type=pl.DeviceIdType.MESH,
            )
        s1a[0, :, :] = produce_attn(my, 0)
        s1b[0, :, :] = produce_attn(my, 1)
        pl.semaphore_wait(barrier_sem, 2)

        for h in range(N_DEV - 1):
            rd_a = mk(h, h, s1a, r1a, ss1a, rs1a, right)
            rd_b = mk(h, h, s1b, r1b, ss1b, rs1b, left)
            rd_a.start()
            rd_b.start()
            ca = lax.rem(my + N_DEV - h - 1, N_DEV)
            cb = lax.rem(my + h + 1, N_DEV)
            pa = produce_attn(ca, 0)
            pb = produce_attn(cb, 1)
            rd_a.wait()
            rd_b.wait()
            acc_a = r1a[h] + pa
            acc_b = r1b[h] + pb
            s1a[h + 1, :, :] = acc_a
            s1b[h + 1, :, :] = acc_b
            if h == N_DEV - 2:
                epilogue1(ca, 0, acc_a)
                epilogue1(cb, 1, acc_b)
                s2a[0, :, :] = produce_ffn(ca, 0)
                s2b[0, :, :] = produce_ffn(cb, 1)

        for g in range(N_DEV - 1):
            h1 = N_DEV - 1 + g
            src1 = 3 if g == 0 else h1
            ag_a = mk(h1, src1, s1a, r1a, ss1a, rs1a, right)
            ag_b = mk(h1, src1, s1b, r1b, ss1b, rs1b, left)
            rs_a = mk(g, g, s2a, r2a, ss2a, rs2a, right)
            rs_b = mk(g, g, s2b, r2b, ss2b, rs2b, left)
            ag_a.start()
            ag_b.start()
            rs_a.start()
            rs_b.start()
            ca = lax.rem(my + N_DEV - g, N_DEV)
            cb = lax.rem(my + g, N_DEV)
            ag_a.wait()
            ag_b.wait()
            epilogue1(ca, 0, r1a[h1])
            epilogue1(cb, 1, r1b[h1])
            pa = produce_ffn(ca, 0)
            pb = produce_ffn(cb, 1)
            rs_a.wait()
            rs_b.wait()
            acc_a = r2a[g] + pa
            acc_b = r2b[g] + pb
            s2a[g + 1, :, :] = acc_a
            s2b[g + 1, :, :] = acc_b
            if g == N_DEV - 2:
                epilogue2(ca, 0, acc_a)
                epilogue2(cb, 1, acc_b)

        for g in range(N_DEV - 1):
            h2 = N_DEV - 1 + g
            src2 = 3 if g == 0 else h2
            ag_a = mk(h2, src2, s2a, r2a, ss2a, rs2a, right)
            ag_b = mk(h2, src2, s2b, r2b, ss2b, rs2b, left)
            ag_a.start()
            ag_b.start()
            ca = lax.rem(my + N_DEV + 1 - g, N_DEV)
            cb = lax.rem(my + 3 + g, N_DEV)
            ag_a.wait()
            ag_b.wait()
            epilogue2(ca, 0, r2a[h2])
            epilogue2(cb, 1, r2b[h2])


    sbuf = lambda: pltpu.VMEM((N_DEV, rows, C), BF16)
    rbuf = lambda: pltpu.VMEM((N_HOPS, rows, C), BF16)
    sem = lambda: pltpu.SemaphoreType.DMA((N_HOPS,))
    return pl.pallas_call(
        body,
        out_shape=jax.ShapeDtypeStruct((R, C), F32),
        in_specs=[pl.BlockSpec(memory_space=pltpu.VMEM)] * 11,
        out_specs=pl.BlockSpec(memory_space=pltpu.VMEM),
        scratch_shapes=[
            pltpu.VMEM((rows, n_heads * Dh), BF16),
            sbuf(), rbuf(), sbuf(), rbuf(),
            sbuf(), rbuf(), sbuf(), rbuf(),
            sem(), sem(), sem(), sem(),
            sem(), sem(), sem(), sem(),
        ],
        compiler_params=pltpu.CompilerParams(
            collective_id=0, vmem_limit_bytes=63 * 1024 * 1024
        ),
    )(Q, K, V, Wo, W1, W2, x, ga, sm, shm, gm)


def kernel(x, Wq, Wk, Wv, Wo, t_emb, W_mod, W_ff1, W_ff2):
    B, S, D = x.shape
    Dh = 128
    Hl = Wq.shape[1] // Dh

    x = x.astype(F32)
    mod = t_emb.astype(F32) @ W_mod.astype(F32)
    sa, sha, ga, sm, shm, gm = jnp.split(mod, 6, axis=-1)

    m = jnp.mean(x, axis=-1, keepdims=True)
    v = jnp.var(x, axis=-1, keepdims=True)
    xa = ((x - m) * lax.rsqrt(v + EPS) * (1.0 + sa[:, None, :])
          + sha[:, None, :]).astype(BF16)
    xa = xa.reshape(B * S, D)

    Q = (xa @ Wq.astype(BF16)).astype(BF16)
    K = (xa @ Wk.astype(BF16)).astype(BF16)
    V = (xa @ Wv.astype(BF16)).astype(BF16)

    out = _dit_block(Q, K, V, Wo.astype(BF16), W_ff1.astype(BF16),
                     W_ff2.astype(BF16), x.reshape(B * S, D),
                     ga, sm, shm, gm, Hl)
    return out.reshape(B, S, D)
